# baseline (speedup 1.0000x reference)
import functools

import jax
import jax.numpy as jnp
from jax import lax
from jax.experimental import pallas as pl
from jax.experimental.pallas import tpu as pltpu
from jax.experimental.pallas import tpu_sc as plsc

V = 100000
EMB_DIM = 32
NC = 2
NS = 16
NW = NC * NS
R = 4096
C = 200
RB = R // NW
V_PER_S = V // NS
SCH = 125
NSC = V_PER_S // SCH

_mesh = plsc.VectorSubcoreMesh(core_axis_name="c", subcore_axis_name="s")


@functools.partial(
    pl.kernel,
    out_type=jax.ShapeDtypeStruct((C, 4, NW, 32, 32), jnp.float32),
    mesh=_mesh,
    scratch_types=[
        pltpu.VMEM_SHARED((V, EMB_DIM), jnp.bfloat16),
        [pltpu.VMEM((RB,), jnp.int32) for _ in range(2)],
        [pltpu.VMEM((RB, EMB_DIM), jnp.bfloat16) for _ in range(2)],
        [pltpu.VMEM((EMB_DIM * (RB + 1),), jnp.float32) for _ in range(2)],
        [pltpu.VMEM((RB, EMB_DIM), jnp.float32) for _ in range(2)],
        [pltpu.SemaphoreType.DMA for _ in range(2)],
        [pltpu.SemaphoreType.DMA for _ in range(2)],
        [pltpu.SemaphoreType.DMA for _ in range(2)],
    ],
    compiler_params=pltpu.CompilerParams(
        use_tc_tiling_on_sc=False, needs_layout_passes=False),
)
def _sc_gather(idx_hbm, tb_hbm, out_hbm, tb_sh, ibufs, bbufs, obufs, sbufs,
               isems, gsems, wsems):
    sid = lax.axis_index("s")
    wid = sid * NC + lax.axis_index("c")
    my_idx = idx_hbm.at[wid]

    tbase = sid * V_PER_S

    def p1_load(k, b):
        pltpu.make_async_copy(
            tb_hbm.at[pl.ds(tbase + k * SCH, SCH)],
            sbufs[b].at[pl.ds(0, SCH)], gsems[b]).start()

    def p1_drain_load(b):
        pltpu.make_async_copy(
            tb_hbm.at[pl.ds(0, SCH)],
            sbufs[b].at[pl.ds(0, SCH)], gsems[b]).wait()

    def p1_conv(b):
        def cbody(r, carry):
            for u in range(5):
                row = r * 5 + u
                a0 = sbufs[b][row, pl.ds(0, 16)]
                a1 = sbufs[b][row, pl.ds(16, 16)]
                bbufs[b][row] = plsc.pack(
                    a0, a1, format=plsc.PackFormat.INTERLEAVED)
            return carry
        lax.fori_loop(0, SCH // 5, cbody, 0)

    def p1_store(k, b):
        pltpu.make_async_copy(
            bbufs[b].at[pl.ds(0, SCH)],
            tb_sh.at[pl.ds(tbase + k * SCH, SCH)], wsems[b]).start()

    def p1_drain_store(b):
        pltpu.make_async_copy(
            bbufs[b].at[pl.ds(0, SCH)],
            tb_sh.at[pl.ds(0, SCH)], wsems[b]).wait()

    p1_load(0, 0)

    def p1_body(i, carry):
        k0 = i * 2
        p1_drain_load(0)
        p1_load(k0 + 1, 1)

        @pl.when(i > 0)
        def _():
            p1_drain_store(0)

        p1_conv(0)
        p1_store(k0, 0)
        p1_drain_load(1)

        @pl.when(i + 1 < NSC // 2)
        def _():
            p1_load(k0 + 2, 0)

        @pl.when(i > 0)
        def _():
            p1_drain_store(1)

        p1_conv(1)
        p1_store(k0 + 1, 1)
        return carry

    lax.fori_loop(0, NSC // 2, p1_body, 0)
    p1_drain_store(0)
    p1_drain_store(1)
    plsc.subcore_barrier()

    lane = lax.iota(jnp.int32, 16)
    kidx0 = lane * (RB + 1)
    kidx1 = kidx0 + 16 * (RB + 1)

    def load_i(c, b):
        pltpu.make_async_copy(my_idx.at[c], ibufs[b], isems[b]).start()

    def drain_i(b):
        pltpu.make_async_copy(my_idx.at[0], ibufs[b], isems[b]).wait()

    def fire_g(b):
        pltpu.make_async_copy(tb_sh.at[ibufs[b]], bbufs[b], gsems[b]).start()

    def drain_g(b):
        pltpu.make_async_copy(
            tb_hbm.at[pl.ds(0, RB)], bbufs[b], gsems[b]).wait()

    def convert(b):
        def cbody(q, carry):
            for u in range(4):
                r = q * 4 + u
                x = bbufs[b][r]
                a0, a1 = plsc.unpack(x, format=plsc.PackFormat.INTERLEAVED)
                plsc.store_scatter(obufs[b], [kidx0 + r], a0)
                plsc.store_scatter(obufs[b], [kidx1 + r], a1)
            return carry
        lax.fori_loop(0, RB // 4, cbody, 0)

        def kbody(k, carry):
            for j in range(8):
                x = plsc.load_gather(
                    obufs[b], [lane + k * (RB + 1) + j * 16])
                sbufs[b][k * 4 + j // 2, pl.ds((j % 2) * 16, 16)] = x
            return carry
        lax.fori_loop(0, EMB_DIM, kbody, 0)

    def wb(c, b):
        for k8 in range(4):
            pltpu.make_async_copy(
                sbufs[b].at[pl.ds(k8 * 32, 32)],
                out_hbm.at[c, k8, wid], wsems[b]).start()

    def drain_w(b):
        for k8 in range(4):
            pltpu.make_async_copy(
                sbufs[b].at[pl.ds(k8 * 32, 32)],
                out_hbm.at[0, k8, 0], wsems[b]).wait()

    load_i(0, 0)
    load_i(1, 1)
    drain_i(0)
    fire_g(0)
    nit = C // 2

    def body(i, carry):
        c0 = i * 2
        drain_i(1)
        fire_g(1)
        drain_g(0)

        @pl.when(i + 1 < nit)
        def _():
            load_i(c0 + 2, 0)

        @pl.when(i > 0)
        def _():
            drain_w(0)

        convert(0)
        wb(c0, 0)
        drain_g(1)

        @pl.when(i + 1 < nit)
        def _():
            load_i(c0 + 3, 1)

        @pl.when(i > 0)
        def _():
            drain_w(1)

        convert(1)
        wb(c0 + 1, 1)

        @pl.when(i + 1 < nit)
        def _():
            drain_i(0)
            fire_g(0)

        return carry

    lax.fori_loop(0, nit, body, 0)
    drain_w(0)
    drain_w(1)


def kernel(timestep, embeddings):
    idx = timestep.astype(jnp.int32).reshape(NW, RB, C).transpose(0, 2, 1)
    out = _sc_gather(idx, embeddings)
    o5 = out.reshape(C, 4, NW, 8, RB)
    return o5.transpose(2, 4, 0, 1, 3).reshape(R, C, EMB_DIM)


# speedup vs baseline: 2.0056x; 2.0056x over previous
import functools

import jax
import jax.numpy as jnp
from jax import lax
from jax.experimental import pallas as pl
from jax.experimental.pallas import tpu as pltpu
from jax.experimental.pallas import tpu_sc as plsc

EMB_DIM = 32
NC = 2
NS = 16
NW = NC * NS
PER_W = 25600
GSZ = 640
CH = 640
NG = CH // GSZ
G = PER_W // CH
NBUF = 4
IDX_ROWS = PER_W // GSZ
N_TOTAL = NW * PER_W

_mesh = plsc.VectorSubcoreMesh(core_axis_name="c", subcore_axis_name="s")


@functools.partial(
    pl.kernel,
    out_type=jax.ShapeDtypeStruct((N_TOTAL, EMB_DIM), jnp.float32),
    mesh=_mesh,
    scratch_types=[
        pltpu.VMEM((IDX_ROWS, GSZ), jnp.int32),
        [pltpu.VMEM((CH, EMB_DIM), jnp.float32) for _ in range(NBUF)],
        [pltpu.SemaphoreType.DMA for _ in range(NBUF)],
        [pltpu.SemaphoreType.DMA for _ in range(NBUF)],
    ],
    compiler_params=pltpu.CompilerParams(use_tc_tiling_on_sc=False),
)
def _sc_gather(idx_hbm, table_hbm, out_hbm, idx_v, bufs, gsems, wsems):
    wid = lax.axis_index("s") * NC + lax.axis_index("c")
    pltpu.sync_copy(idx_hbm.at[wid], idx_v)
    out_base = wid * PER_W

    def fire(c, b):
        for j in range(NG):
            pltpu.make_async_copy(
                table_hbm.at[idx_v.at[c * NG + j]],
                bufs[b].at[pl.ds(j * GSZ, GSZ)],
                gsems[b],
            ).start()

    def drain_g(b):
        pltpu.make_async_copy(
            out_hbm.at[pl.ds(0, CH)], bufs[b], gsems[b]).wait()

    def wb(c, b):
        pltpu.make_async_copy(
            bufs[b], out_hbm.at[pl.ds(out_base + c * CH, CH)], wsems[b]
        ).start()

    def drain_w(b):
        pltpu.make_async_copy(
            bufs[b], out_hbm.at[pl.ds(0, CH)], wsems[b]).wait()

    for b in range(NBUF):
        fire(b, b)

    nit = G // NBUF

    def body(i, carry):
        c0 = i * NBUF
        for b in range(NBUF):
            drain_g(b)
            wb(c0 + b, b)

        @pl.when(i + 1 < nit)
        def _():
            for b in range(NBUF):
                drain_w(b)
                fire(c0 + NBUF + b, b)

        return carry

    lax.fori_loop(0, nit, body, 0)
    for b in range(NBUF):
        drain_w(b)


def kernel(timestep, embeddings):
    idx = timestep.reshape(-1).astype(jnp.int32)
    idx = idx.reshape(NW, IDX_ROWS, GSZ)
    out = _sc_gather(idx, embeddings)
    return out.reshape(timestep.shape + (EMB_DIM,))
